# Initial kernel scaffold; baseline (speedup 1.0000x reference)
import jax
import jax.numpy as jnp
from jax.experimental import pallas as pl


def kernel(edge_index, joint_acts, edge_feats_u, node_feats_u, edge_feat_reflected_u, W_ju1, b_ju1, W_ju3, b_ju3, W_ju2, b_ju2, W_iu1, b_iu1, W_iu3, b_iu3, W_iu2, b_iu2):
    raise NotImplementedError("write your pallas kernel here")


# speedup vs baseline: 4.6689x; 4.6689x over previous
import functools

import jax
import jax.numpy as jnp
from jax import lax
from jax.experimental import pallas as pl
from jax.experimental.pallas import tpu as pltpu
from jax.experimental.pallas import tpu_sc as plsc

_N = 10000
_E = 160000
_A = 8
_DIM = 128

_NUM_WORKERS = 32
_CHUNK = 5008
_EPAD = _NUM_WORKERS * _CHUNK

_BE = 2000


def _sc_codes(joint_acts, src_pad, dst_pad):
    mesh = plsc.VectorSubcoreMesh(core_axis_name="c", subcore_axis_name="s")

    @functools.partial(
        pl.kernel,
        mesh=mesh,
        compiler_params=pltpu.CompilerParams(needs_layout_passes=False),
        out_type=[
            jax.ShapeDtypeStruct((_EPAD,), jnp.int32),
            jax.ShapeDtypeStruct((_EPAD,), jnp.int32),
        ],
        scratch_types=[
            pltpu.VMEM((_N,), jnp.int32),
            pltpu.VMEM((_CHUNK,), jnp.int32),
            pltpu.VMEM((_CHUNK,), jnp.int32),
            pltpu.VMEM((_CHUNK,), jnp.int32),
            pltpu.VMEM((_CHUNK,), jnp.int32),
        ],
    )
    def k(ja_hbm, src_hbm, dst_hbm, c_hbm, cr_hbm, ja_v, src_v, dst_v, c_v, cr_v):
        wid = lax.axis_index("s") * 2 + lax.axis_index("c")
        base = wid * _CHUNK
        pltpu.sync_copy(ja_hbm, ja_v)
        pltpu.sync_copy(src_hbm.at[pl.ds(base, _CHUNK)], src_v)
        pltpu.sync_copy(dst_hbm.at[pl.ds(base, _CHUNK)], dst_v)

        def body(i, carry):
            sl = pl.ds(i * 16, 16)
            a_s = plsc.load_gather(ja_v, [src_v[sl]])
            a_d = plsc.load_gather(ja_v, [dst_v[sl]])
            c_v[sl] = a_s * _A + a_d
            cr_v[sl] = a_d * _A + a_s
            return carry

        lax.fori_loop(0, _CHUNK // 16, body, 0, unroll=4)
        pltpu.sync_copy(c_v, c_hbm.at[pl.ds(base, _CHUNK)])
        pltpu.sync_copy(cr_v, cr_hbm.at[pl.ds(base, _CHUNK)])

    return k(joint_acts, src_pad, dst_pad)


def _edge_body(xu_ref, xr_ref, c_ref, cr_ref,
               w1_ref, b1_ref, w3_ref, b3_ref, w2_ref, b2_ref, out_ref):
    @pl.when(pl.program_id(0) == 0)
    def _init():
        out_ref[...] = jnp.zeros_like(out_ref)

    def stream(x, idx_col):
        g = jnp.maximum(
            jnp.dot(x, w1_ref[...], preferred_element_type=jnp.float32)
            + b1_ref[...], 0.0)
        g = jnp.maximum(
            jnp.dot(g, w3_ref[...], preferred_element_type=jnp.float32)
            + b3_ref[...], 0.0)
        logit = (jnp.dot(g, w2_ref[...], preferred_element_type=jnp.float32)
                 + b2_ref[...])
        onehot = idx_col == lax.broadcasted_iota(jnp.int32, logit.shape, 1)
        return jnp.sum(jnp.where(onehot, logit, 0.0))

    part = stream(xu_ref[...], c_ref[...]) + stream(xr_ref[...], cr_ref[...])
    out_ref[...] = out_ref[...] + part


def _node_body(x_ref, ja_ref, w1_ref, b1_ref, w3_ref, b3_ref, w2_ref, b2_ref,
               out_ref):
    h = jnp.maximum(
        jnp.dot(x_ref[...], w1_ref[...], preferred_element_type=jnp.float32)
        + b1_ref[...], 0.0)
    h = jnp.maximum(
        jnp.dot(h, w3_ref[...], preferred_element_type=jnp.float32)
        + b3_ref[...], 0.0)
    logit = (jnp.dot(h, w2_ref[...], preferred_element_type=jnp.float32)
             + b2_ref[...])
    onehot = ja_ref[...] == lax.broadcasted_iota(jnp.int32, logit.shape, 1)
    out_ref[...] = jnp.sum(jnp.where(onehot, logit, 0.0)).reshape(1, 1)


def kernel(edge_index, joint_acts, edge_feats_u, node_feats_u,
           edge_feat_reflected_u, W_ju1, b_ju1, W_ju3, b_ju3, W_ju2, b_ju2,
           W_iu1, b_iu1, W_iu3, b_iu3, W_iu2, b_iu2):
    src = edge_index[0]
    dst = edge_index[1]
    pad = jnp.zeros((_EPAD - _E,), jnp.int32)
    c_pad, cr_pad = _sc_codes(joint_acts,
                              jnp.concatenate([src, pad]),
                              jnp.concatenate([dst, pad]))
    c_col = c_pad[:_E].reshape(_E, 1)
    cr_col = cr_pad[:_E].reshape(_E, 1)

    wfull = lambda shape: pl.BlockSpec(shape, lambda i: (0,) * len(shape))
    edge_out = pl.pallas_call(
        _edge_body,
        grid=(_E // _BE,),
        in_specs=[
            pl.BlockSpec((_BE, 3 * _DIM), lambda i: (i, 0)),
            pl.BlockSpec((_BE, 3 * _DIM), lambda i: (i, 0)),
            pl.BlockSpec((_BE, 1), lambda i: (i, 0)),
            pl.BlockSpec((_BE, 1), lambda i: (i, 0)),
            wfull((3 * _DIM, 32)),
            wfull((1, 32)),
            wfull((32, 32)),
            wfull((1, 32)),
            wfull((32, _A * _A)),
            wfull((1, _A * _A)),
        ],
        out_specs=pl.BlockSpec((1, 1), lambda i: (0, 0)),
        out_shape=jax.ShapeDtypeStruct((1, 1), jnp.float32),
    )(edge_feats_u, edge_feat_reflected_u, c_col, cr_col,
      W_ju1, b_ju1.reshape(1, -1), W_ju3, b_ju3.reshape(1, -1),
      W_ju2, b_ju2.reshape(1, -1))

    node_out = pl.pallas_call(
        _node_body,
        grid=(1,),
        in_specs=[
            pl.BlockSpec((_N, 2 * _DIM), lambda i: (0, 0)),
            pl.BlockSpec((_N, 1), lambda i: (0, 0)),
            wfull((2 * _DIM, 32)),
            wfull((1, 32)),
            wfull((32, 32)),
            wfull((1, 32)),
            wfull((32, _A)),
            wfull((1, _A)),
        ],
        out_specs=pl.BlockSpec((1, 1), lambda i: (0, 0)),
        out_shape=jax.ShapeDtypeStruct((1, 1), jnp.float32),
    )(node_feats_u, joint_acts.reshape(_N, 1),
      W_iu1, b_iu1.reshape(1, -1), W_iu3, b_iu3.reshape(1, -1),
      W_iu2, b_iu2.reshape(1, -1))

    return node_out + 0.25 * edge_out


# speedup vs baseline: 4.8868x; 1.0467x over previous
import functools

import jax
import jax.numpy as jnp
from jax import lax
from jax.experimental import pallas as pl
from jax.experimental.pallas import tpu as pltpu
from jax.experimental.pallas import tpu_sc as plsc

_N = 10000
_E = 160000
_A = 8
_DIM = 128

_NUM_WORKERS = 32
_CHUNK = 5008
_EPAD = _NUM_WORKERS * _CHUNK

_BE = 4000


def _sc_codes(joint_acts, src_pad, dst_pad):
    mesh = plsc.VectorSubcoreMesh(core_axis_name="c", subcore_axis_name="s")

    @functools.partial(
        pl.kernel,
        mesh=mesh,
        compiler_params=pltpu.CompilerParams(needs_layout_passes=False),
        out_type=[
            jax.ShapeDtypeStruct((_EPAD,), jnp.int32),
            jax.ShapeDtypeStruct((_EPAD,), jnp.int32),
        ],
        scratch_types=[
            pltpu.VMEM((_N,), jnp.int32),
            pltpu.VMEM((_CHUNK,), jnp.int32),
            pltpu.VMEM((_CHUNK,), jnp.int32),
            pltpu.VMEM((_CHUNK,), jnp.int32),
            pltpu.VMEM((_CHUNK,), jnp.int32),
        ],
    )
    def k(ja_hbm, src_hbm, dst_hbm, c_hbm, cr_hbm, ja_v, src_v, dst_v, c_v, cr_v):
        wid = lax.axis_index("s") * 2 + lax.axis_index("c")
        base = wid * _CHUNK
        pltpu.sync_copy(ja_hbm, ja_v)
        pltpu.sync_copy(src_hbm.at[pl.ds(base, _CHUNK)], src_v)
        pltpu.sync_copy(dst_hbm.at[pl.ds(base, _CHUNK)], dst_v)

        def body(i, carry):
            sl = pl.ds(i * 16, 16)
            a_s = plsc.load_gather(ja_v, [src_v[sl]])
            a_d = plsc.load_gather(ja_v, [dst_v[sl]])
            c_v[sl] = a_s * _A + a_d
            cr_v[sl] = a_d * _A + a_s
            return carry

        lax.fori_loop(0, _CHUNK // 16, body, 0, unroll=4)
        pltpu.sync_copy(c_v, c_hbm.at[pl.ds(base, _CHUNK)])
        pltpu.sync_copy(cr_v, cr_hbm.at[pl.ds(base, _CHUNK)])

    return k(joint_acts, src_pad, dst_pad)


def _edge_body(xu_ref, xr_ref, c_ref, cr_ref,
               w1_ref, b1_ref, w3_ref, b3_ref, w2_ref, b2_ref, out_ref):
    @pl.when(pl.program_id(0) == 0)
    def _init():
        out_ref[...] = jnp.zeros_like(out_ref)

    def stream(x, idx_col):
        g = jnp.maximum(
            jnp.dot(x, w1_ref[...], preferred_element_type=jnp.float32)
            + b1_ref[...], 0.0)
        g = jnp.maximum(
            jnp.dot(g, w3_ref[...], preferred_element_type=jnp.float32)
            + b3_ref[...], 0.0)
        logit = (jnp.dot(g, w2_ref[...], preferred_element_type=jnp.float32)
                 + b2_ref[...])
        onehot = idx_col == lax.broadcasted_iota(jnp.int32, logit.shape, 1)
        return jnp.sum(jnp.where(onehot, logit, 0.0))

    part = stream(xu_ref[...], c_ref[...]) + stream(xr_ref[...], cr_ref[...])
    out_ref[...] = out_ref[...] + part


def _node_body(x_ref, ja_ref, w1_ref, b1_ref, w3_ref, b3_ref, w2_ref, b2_ref,
               out_ref):
    h = jnp.maximum(
        jnp.dot(x_ref[...], w1_ref[...], preferred_element_type=jnp.float32)
        + b1_ref[...], 0.0)
    h = jnp.maximum(
        jnp.dot(h, w3_ref[...], preferred_element_type=jnp.float32)
        + b3_ref[...], 0.0)
    logit = (jnp.dot(h, w2_ref[...], preferred_element_type=jnp.float32)
             + b2_ref[...])
    onehot = ja_ref[...] == lax.broadcasted_iota(jnp.int32, logit.shape, 1)
    out_ref[...] = jnp.sum(jnp.where(onehot, logit, 0.0)).reshape(1, 1)


def kernel(edge_index, joint_acts, edge_feats_u, node_feats_u,
           edge_feat_reflected_u, W_ju1, b_ju1, W_ju3, b_ju3, W_ju2, b_ju2,
           W_iu1, b_iu1, W_iu3, b_iu3, W_iu2, b_iu2):
    src = edge_index[0]
    dst = edge_index[1]
    pad = jnp.zeros((_EPAD - _E,), jnp.int32)
    c_pad, cr_pad = _sc_codes(joint_acts,
                              jnp.concatenate([src, pad]),
                              jnp.concatenate([dst, pad]))
    c_col = c_pad[:_E].reshape(_E, 1)
    cr_col = cr_pad[:_E].reshape(_E, 1)

    wfull = lambda shape: pl.BlockSpec(shape, lambda i: (0,) * len(shape))
    edge_out = pl.pallas_call(
        _edge_body,
        grid=(_E // _BE,),
        in_specs=[
            pl.BlockSpec((_BE, 3 * _DIM), lambda i: (i, 0)),
            pl.BlockSpec((_BE, 3 * _DIM), lambda i: (i, 0)),
            pl.BlockSpec((_BE, 1), lambda i: (i, 0)),
            pl.BlockSpec((_BE, 1), lambda i: (i, 0)),
            wfull((3 * _DIM, 32)),
            wfull((1, 32)),
            wfull((32, 32)),
            wfull((1, 32)),
            wfull((32, _A * _A)),
            wfull((1, _A * _A)),
        ],
        out_specs=pl.BlockSpec((1, 1), lambda i: (0, 0)),
        out_shape=jax.ShapeDtypeStruct((1, 1), jnp.float32),
    )(edge_feats_u, edge_feat_reflected_u, c_col, cr_col,
      W_ju1, b_ju1.reshape(1, -1), W_ju3, b_ju3.reshape(1, -1),
      W_ju2, b_ju2.reshape(1, -1))

    node_out = pl.pallas_call(
        _node_body,
        grid=(1,),
        in_specs=[
            pl.BlockSpec((_N, 2 * _DIM), lambda i: (0, 0)),
            pl.BlockSpec((_N, 1), lambda i: (0, 0)),
            wfull((2 * _DIM, 32)),
            wfull((1, 32)),
            wfull((32, 32)),
            wfull((1, 32)),
            wfull((32, _A)),
            wfull((1, _A)),
        ],
        out_specs=pl.BlockSpec((1, 1), lambda i: (0, 0)),
        out_shape=jax.ShapeDtypeStruct((1, 1), jnp.float32),
    )(node_feats_u, joint_acts.reshape(_N, 1),
      W_iu1, b_iu1.reshape(1, -1), W_iu3, b_iu3.reshape(1, -1),
      W_iu2, b_iu2.reshape(1, -1))

    return node_out + 0.25 * edge_out


# speedup vs baseline: 8.1834x; 1.6746x over previous
import functools

import jax
import jax.numpy as jnp
from jax import lax
from jax.experimental import pallas as pl
from jax.experimental.pallas import tpu as pltpu
from jax.experimental.pallas import tpu_sc as plsc

_N = 10000
_E = 160000
_A = 8
_DIM = 128

_NUM_WORKERS = 32
_CHUNK = 5008
_EPAD = _NUM_WORKERS * _CHUNK

_BE = 4000


def _sc_codes(joint_acts, src_pad, dst_pad):
    mesh = plsc.VectorSubcoreMesh(core_axis_name="c", subcore_axis_name="s")

    @functools.partial(
        pl.kernel,
        mesh=mesh,
        compiler_params=pltpu.CompilerParams(needs_layout_passes=False),
        out_type=[
            jax.ShapeDtypeStruct((_EPAD,), jnp.int32),
            jax.ShapeDtypeStruct((_EPAD,), jnp.int32),
        ],
        scratch_types=[
            pltpu.VMEM((_N,), jnp.int32),
            pltpu.VMEM((_CHUNK,), jnp.int32),
            pltpu.VMEM((_CHUNK,), jnp.int32),
            pltpu.VMEM((_CHUNK,), jnp.int32),
            pltpu.VMEM((_CHUNK,), jnp.int32),
        ],
    )
    def k(ja_hbm, src_hbm, dst_hbm, c_hbm, cr_hbm, ja_v, src_v, dst_v, c_v, cr_v):
        wid = lax.axis_index("s") * 2 + lax.axis_index("c")
        base = wid * _CHUNK
        pltpu.sync_copy(ja_hbm, ja_v)
        pltpu.sync_copy(src_hbm.at[pl.ds(base, _CHUNK)], src_v)
        pltpu.sync_copy(dst_hbm.at[pl.ds(base, _CHUNK)], dst_v)

        def body(i, carry):
            sl = pl.ds(i * 16, 16)
            a_s = plsc.load_gather(ja_v, [src_v[sl]])
            a_d = plsc.load_gather(ja_v, [dst_v[sl]])
            c_v[sl] = a_s * _A + a_d
            cr_v[sl] = a_d * _A + a_s
            return carry

        lax.fori_loop(0, _CHUNK // 16, body, 0, unroll=4)
        pltpu.sync_copy(c_v, c_hbm.at[pl.ds(base, _CHUNK)])
        pltpu.sync_copy(cr_v, cr_hbm.at[pl.ds(base, _CHUNK)])

    return k(joint_acts, src_pad, dst_pad)


def _edge_body(xu_ref, xr_ref, c_ref, cr_ref,
               w1_ref, b1_ref, w3_ref, b3_ref, w2_ref, b2_ref, out_ref):
    @pl.when(pl.program_id(0) == 0)
    def _init():
        out_ref[...] = jnp.zeros_like(out_ref)

    diag = (lax.broadcasted_iota(jnp.int32, (_A * _A, _A * _A), 0)
            == lax.broadcasted_iota(jnp.int32, (_A * _A, _A * _A), 1))

    def stream(x, idx_row):
        g = jnp.maximum(
            jnp.dot(x, w1_ref[...], preferred_element_type=jnp.float32)
            + b1_ref[...], 0.0)
        g = jnp.maximum(
            jnp.dot(g, w3_ref[...], preferred_element_type=jnp.float32)
            + b3_ref[...], 0.0)
        logit = (jnp.dot(g, w2_ref[...], preferred_element_type=jnp.float32)
                 + b2_ref[...])
        onehot_t = jnp.where(
            idx_row == lax.broadcasted_iota(jnp.int32, (_A * _A, _BE), 0),
            1.0, 0.0)
        prod = jnp.dot(onehot_t, logit, preferred_element_type=jnp.float32)
        return jnp.sum(jnp.where(diag, prod, 0.0))

    part = (stream(xu_ref[...], c_ref[...].reshape(1, _BE))
            + stream(xr_ref[...], cr_ref[...].reshape(1, _BE)))
    out_ref[...] = out_ref[...] + part


def _node_body(x_ref, ja_ref, w1_ref, b1_ref, w3_ref, b3_ref, w2_ref, b2_ref,
               out_ref):
    h = jnp.maximum(
        jnp.dot(x_ref[...], w1_ref[...], preferred_element_type=jnp.float32)
        + b1_ref[...], 0.0)
    h = jnp.maximum(
        jnp.dot(h, w3_ref[...], preferred_element_type=jnp.float32)
        + b3_ref[...], 0.0)
    logit = (jnp.dot(h, w2_ref[...], preferred_element_type=jnp.float32)
             + b2_ref[...])
    onehot_t = jnp.where(
        ja_ref[...].reshape(1, _N)
        == lax.broadcasted_iota(jnp.int32, (_A, _N), 0), 1.0, 0.0)
    prod = jnp.dot(onehot_t, logit, preferred_element_type=jnp.float32)
    diag = (lax.broadcasted_iota(jnp.int32, (_A, _A), 0)
            == lax.broadcasted_iota(jnp.int32, (_A, _A), 1))
    out_ref[...] = jnp.sum(jnp.where(diag, prod, 0.0)).reshape(1, 1)


def kernel(edge_index, joint_acts, edge_feats_u, node_feats_u,
           edge_feat_reflected_u, W_ju1, b_ju1, W_ju3, b_ju3, W_ju2, b_ju2,
           W_iu1, b_iu1, W_iu3, b_iu3, W_iu2, b_iu2):
    src = edge_index[0]
    dst = edge_index[1]
    pad = jnp.zeros((_EPAD - _E,), jnp.int32)
    c_pad, cr_pad = _sc_codes(joint_acts,
                              jnp.concatenate([src, pad]),
                              jnp.concatenate([dst, pad]))
    nb = _E // _BE
    c_rows = c_pad[:_E].reshape(nb, 1, _BE)
    cr_rows = cr_pad[:_E].reshape(nb, 1, _BE)

    wfull = lambda shape: pl.BlockSpec(shape, lambda i: (0,) * len(shape))
    edge_out = pl.pallas_call(
        _edge_body,
        grid=(_E // _BE,),
        in_specs=[
            pl.BlockSpec((_BE, 3 * _DIM), lambda i: (i, 0)),
            pl.BlockSpec((_BE, 3 * _DIM), lambda i: (i, 0)),
            pl.BlockSpec((1, 1, _BE), lambda i: (i, 0, 0)),
            pl.BlockSpec((1, 1, _BE), lambda i: (i, 0, 0)),
            wfull((3 * _DIM, 32)),
            wfull((1, 32)),
            wfull((32, 32)),
            wfull((1, 32)),
            wfull((32, _A * _A)),
            wfull((1, _A * _A)),
        ],
        out_specs=pl.BlockSpec((1, 1), lambda i: (0, 0)),
        out_shape=jax.ShapeDtypeStruct((1, 1), jnp.float32),
    )(edge_feats_u, edge_feat_reflected_u, c_rows, cr_rows,
      W_ju1, b_ju1.reshape(1, -1), W_ju3, b_ju3.reshape(1, -1),
      W_ju2, b_ju2.reshape(1, -1))

    node_out = pl.pallas_call(
        _node_body,
        grid=(1,),
        in_specs=[
            pl.BlockSpec((_N, 2 * _DIM), lambda i: (0, 0)),
            pl.BlockSpec((1, 1, _N), lambda i: (0, 0, 0)),
            wfull((2 * _DIM, 32)),
            wfull((1, 32)),
            wfull((32, 32)),
            wfull((1, 32)),
            wfull((32, _A)),
            wfull((1, _A)),
        ],
        out_specs=pl.BlockSpec((1, 1), lambda i: (0, 0)),
        out_shape=jax.ShapeDtypeStruct((1, 1), jnp.float32),
    )(node_feats_u, joint_acts.reshape(1, 1, _N),
      W_iu1, b_iu1.reshape(1, -1), W_iu3, b_iu3.reshape(1, -1),
      W_iu2, b_iu2.reshape(1, -1))

    return node_out + 0.25 * edge_out


# speedup vs baseline: 8.6846x; 1.0612x over previous
import functools

import jax
import jax.numpy as jnp
from jax import lax
from jax.experimental import pallas as pl
from jax.experimental.pallas import tpu as pltpu
from jax.experimental.pallas import tpu_sc as plsc

_N = 10000
_E = 160000
_A = 8
_DIM = 128

_NUM_WORKERS = 32
_CHUNK = 5008
_EPAD = _NUM_WORKERS * _CHUNK

_BE = 4000


def _sc_codes(joint_acts, src_pad, dst_pad):
    mesh = plsc.VectorSubcoreMesh(core_axis_name="c", subcore_axis_name="s")

    @functools.partial(
        pl.kernel,
        mesh=mesh,
        compiler_params=pltpu.CompilerParams(needs_layout_passes=False),
        out_type=[
            jax.ShapeDtypeStruct((_EPAD,), jnp.int32),
            jax.ShapeDtypeStruct((_EPAD,), jnp.int32),
        ],
        scratch_types=[
            pltpu.VMEM((_N,), jnp.int32),
            pltpu.VMEM((_CHUNK,), jnp.int32),
            pltpu.VMEM((_CHUNK,), jnp.int32),
            pltpu.VMEM((_CHUNK,), jnp.int32),
            pltpu.VMEM((_CHUNK,), jnp.int32),
        ],
    )
    def k(ja_hbm, src_hbm, dst_hbm, c_hbm, cr_hbm, ja_v, src_v, dst_v, c_v, cr_v):
        wid = lax.axis_index("s") * 2 + lax.axis_index("c")
        base = wid * _CHUNK
        pltpu.sync_copy(ja_hbm, ja_v)
        pltpu.sync_copy(src_hbm.at[pl.ds(base, _CHUNK)], src_v)
        pltpu.sync_copy(dst_hbm.at[pl.ds(base, _CHUNK)], dst_v)

        def body(i, carry):
            sl = pl.ds(i * 16, 16)
            a_s = plsc.load_gather(ja_v, [src_v[sl]])
            a_d = plsc.load_gather(ja_v, [dst_v[sl]])
            c_v[sl] = a_s * _A + a_d
            cr_v[sl] = a_d * _A + a_s
            return carry

        lax.fori_loop(0, _CHUNK // 16, body, 0, unroll=4)
        pltpu.sync_copy(c_v, c_hbm.at[pl.ds(base, _CHUNK)])
        pltpu.sync_copy(cr_v, cr_hbm.at[pl.ds(base, _CHUNK)])

    return k(joint_acts, src_pad, dst_pad)


def _edge_body(xu_ref, xr_ref, c_ref, cr_ref,
               w1_ref, b1_ref, w3_ref, b3_ref, w2_ref, b2t_ref, out_ref):
    @pl.when(pl.program_id(0) == 0)
    def _init():
        out_ref[...] = jnp.zeros_like(out_ref)

    diag = (lax.broadcasted_iota(jnp.int32, (_A * _A, _A * _A), 0)
            == lax.broadcasted_iota(jnp.int32, (_A * _A, _A * _A), 1))

    def stream(x, idx_row):
        g = jnp.maximum(
            jnp.dot(x, w1_ref[...], preferred_element_type=jnp.float32)
            + b1_ref[...], 0.0)
        g = jnp.maximum(
            jnp.dot(g, w3_ref[...], preferred_element_type=jnp.float32)
            + b3_ref[...], 0.0)
        onehot_t = jnp.where(
            idx_row == lax.broadcasted_iota(jnp.int32, (_A * _A, _BE), 0),
            1.0, 0.0)
        og = jnp.dot(onehot_t, g, preferred_element_type=jnp.float32)
        prod = jnp.dot(og, w2_ref[...], preferred_element_type=jnp.float32)
        return (jnp.sum(jnp.where(diag, prod, 0.0))
                + jnp.sum(onehot_t * b2t_ref[...]))

    part = (stream(xu_ref[...], c_ref[...].reshape(1, _BE))
            + stream(xr_ref[...], cr_ref[...].reshape(1, _BE)))
    out_ref[...] = out_ref[...] + part


def _node_body(x_ref, ja_ref, w1_ref, b1_ref, w3_ref, b3_ref, w2_ref, b2_ref,
               out_ref):
    h = jnp.maximum(
        jnp.dot(x_ref[...], w1_ref[...], preferred_element_type=jnp.float32)
        + b1_ref[...], 0.0)
    h = jnp.maximum(
        jnp.dot(h, w3_ref[...], preferred_element_type=jnp.float32)
        + b3_ref[...], 0.0)
    logit = (jnp.dot(h, w2_ref[...], preferred_element_type=jnp.float32)
             + b2_ref[...])
    onehot_t = jnp.where(
        ja_ref[...].reshape(1, _N)
        == lax.broadcasted_iota(jnp.int32, (_A, _N), 0), 1.0, 0.0)
    prod = jnp.dot(onehot_t, logit, preferred_element_type=jnp.float32)
    diag = (lax.broadcasted_iota(jnp.int32, (_A, _A), 0)
            == lax.broadcasted_iota(jnp.int32, (_A, _A), 1))
    out_ref[...] = jnp.sum(jnp.where(diag, prod, 0.0)).reshape(1, 1)


def kernel(edge_index, joint_acts, edge_feats_u, node_feats_u,
           edge_feat_reflected_u, W_ju1, b_ju1, W_ju3, b_ju3, W_ju2, b_ju2,
           W_iu1, b_iu1, W_iu3, b_iu3, W_iu2, b_iu2):
    src = edge_index[0]
    dst = edge_index[1]
    pad = jnp.zeros((_EPAD - _E,), jnp.int32)
    c_pad, cr_pad = _sc_codes(joint_acts,
                              jnp.concatenate([src, pad]),
                              jnp.concatenate([dst, pad]))
    nb = _E // _BE
    c_rows = c_pad[:_E].reshape(nb, 1, _BE)
    cr_rows = cr_pad[:_E].reshape(nb, 1, _BE)

    wfull = lambda shape: pl.BlockSpec(shape, lambda i: (0,) * len(shape))
    edge_out = pl.pallas_call(
        _edge_body,
        grid=(_E // _BE,),
        in_specs=[
            pl.BlockSpec((_BE, 3 * _DIM), lambda i: (i, 0)),
            pl.BlockSpec((_BE, 3 * _DIM), lambda i: (i, 0)),
            pl.BlockSpec((1, 1, _BE), lambda i: (i, 0, 0)),
            pl.BlockSpec((1, 1, _BE), lambda i: (i, 0, 0)),
            wfull((3 * _DIM, 32)),
            wfull((1, 32)),
            wfull((32, 32)),
            wfull((1, 32)),
            wfull((32, _A * _A)),
            wfull((_A * _A, 1)),
        ],
        out_specs=pl.BlockSpec((1, 1), lambda i: (0, 0)),
        out_shape=jax.ShapeDtypeStruct((1, 1), jnp.float32),
    )(edge_feats_u, edge_feat_reflected_u, c_rows, cr_rows,
      W_ju1, b_ju1.reshape(1, -1), W_ju3, b_ju3.reshape(1, -1),
      W_ju2, b_ju2.reshape(-1, 1))

    node_out = pl.pallas_call(
        _node_body,
        grid=(1,),
        in_specs=[
            pl.BlockSpec((_N, 2 * _DIM), lambda i: (0, 0)),
            pl.BlockSpec((1, 1, _N), lambda i: (0, 0, 0)),
            wfull((2 * _DIM, 32)),
            wfull((1, 32)),
            wfull((32, 32)),
            wfull((1, 32)),
            wfull((32, _A)),
            wfull((1, _A)),
        ],
        out_specs=pl.BlockSpec((1, 1), lambda i: (0, 0)),
        out_shape=jax.ShapeDtypeStruct((1, 1), jnp.float32),
    )(node_feats_u, joint_acts.reshape(1, 1, _N),
      W_iu1, b_iu1.reshape(1, -1), W_iu3, b_iu3.reshape(1, -1),
      W_iu2, b_iu2.reshape(1, -1))

    return node_out + 0.25 * edge_out


# speedup vs baseline: 8.9628x; 1.0320x over previous
import functools

import jax
import jax.numpy as jnp
from jax import lax
from jax.experimental import pallas as pl
from jax.experimental.pallas import tpu as pltpu
from jax.experimental.pallas import tpu_sc as plsc

_N = 10000
_E = 160000
_A = 8
_DIM = 128

_NUM_WORKERS = 32
_CHUNK = _E // _NUM_WORKERS
_CPAD = (_CHUNK + 15) // 16 * 16

_BE = 4000


def _sc_codes(joint_acts, edge_index):
    mesh = plsc.VectorSubcoreMesh(core_axis_name="c", subcore_axis_name="s")

    @functools.partial(
        pl.kernel,
        mesh=mesh,
        compiler_params=pltpu.CompilerParams(needs_layout_passes=False),
        out_type=[
            jax.ShapeDtypeStruct((_E,), jnp.int32),
            jax.ShapeDtypeStruct((_E,), jnp.int32),
        ],
        scratch_types=[
            pltpu.VMEM((_N,), jnp.int32),
            pltpu.VMEM((_CPAD,), jnp.int32),
            pltpu.VMEM((_CPAD,), jnp.int32),
            pltpu.VMEM((_CPAD,), jnp.int32),
            pltpu.VMEM((_CPAD,), jnp.int32),
        ],
    )
    def k(ja_hbm, ei_hbm, c_hbm, cr_hbm, ja_v, src_v, dst_v, c_v, cr_v):
        wid = lax.axis_index("s") * 2 + lax.axis_index("c")
        base = wid * _CHUNK
        src_v[pl.ds(_CPAD - 16, 16)] = jnp.zeros((16,), jnp.int32)
        dst_v[pl.ds(_CPAD - 16, 16)] = jnp.zeros((16,), jnp.int32)
        pltpu.sync_copy(ja_hbm, ja_v)
        pltpu.sync_copy(ei_hbm.at[pl.ds(base, _CHUNK)], src_v.at[pl.ds(0, _CHUNK)])
        pltpu.sync_copy(ei_hbm.at[pl.ds(_E + base, _CHUNK)], dst_v.at[pl.ds(0, _CHUNK)])

        def body(i, carry):
            sl = pl.ds(i * 16, 16)
            a_s = plsc.load_gather(ja_v, [src_v[sl]])
            a_d = plsc.load_gather(ja_v, [dst_v[sl]])
            c_v[sl] = a_s * _A + a_d
            cr_v[sl] = a_d * _A + a_s
            return carry

        lax.fori_loop(0, _CPAD // 16, body, 0, unroll=4)
        pltpu.sync_copy(c_v.at[pl.ds(0, _CHUNK)], c_hbm.at[pl.ds(base, _CHUNK)])
        pltpu.sync_copy(cr_v.at[pl.ds(0, _CHUNK)], cr_hbm.at[pl.ds(base, _CHUNK)])

    return k(joint_acts, edge_index.reshape(2 * _E))


def _edge_body(xu_ref, xr_ref, c_ref, cr_ref,
               w1_ref, b1_ref, w3_ref, b3_ref, w2_ref, b2t_ref, out_ref):
    @pl.when(pl.program_id(0) == 0)
    def _init():
        out_ref[...] = jnp.zeros_like(out_ref)

    diag = (lax.broadcasted_iota(jnp.int32, (_A * _A, _A * _A), 0)
            == lax.broadcasted_iota(jnp.int32, (_A * _A, _A * _A), 1))

    def stream(x, idx_row):
        g = jnp.maximum(
            jnp.dot(x, w1_ref[...], preferred_element_type=jnp.float32)
            + b1_ref[...], 0.0)
        g = jnp.maximum(
            jnp.dot(g, w3_ref[...], preferred_element_type=jnp.float32)
            + b3_ref[...], 0.0)
        onehot_t = jnp.where(
            idx_row == lax.broadcasted_iota(jnp.int32, (_A * _A, _BE), 0),
            1.0, 0.0)
        og = jnp.dot(onehot_t, g, preferred_element_type=jnp.float32)
        prod = jnp.dot(og, w2_ref[...], preferred_element_type=jnp.float32)
        return (jnp.sum(jnp.where(diag, prod, 0.0))
                + jnp.sum(onehot_t * b2t_ref[...]))

    part = (stream(xu_ref[...], c_ref[...].reshape(1, _BE))
            + stream(xr_ref[...], cr_ref[...].reshape(1, _BE)))
    out_ref[...] = out_ref[...] + part


def _node_body(x_ref, ja_ref, w1_ref, b1_ref, w3_ref, b3_ref, w2_ref, b2_ref,
               out_ref):
    h = jnp.maximum(
        jnp.dot(x_ref[...], w1_ref[...], preferred_element_type=jnp.float32)
        + b1_ref[...], 0.0)
    h = jnp.maximum(
        jnp.dot(h, w3_ref[...], preferred_element_type=jnp.float32)
        + b3_ref[...], 0.0)
    logit = (jnp.dot(h, w2_ref[...], preferred_element_type=jnp.float32)
             + b2_ref[...])
    onehot_t = jnp.where(
        ja_ref[...].reshape(1, _N)
        == lax.broadcasted_iota(jnp.int32, (_A, _N), 0), 1.0, 0.0)
    prod = jnp.dot(onehot_t, logit, preferred_element_type=jnp.float32)
    diag = (lax.broadcasted_iota(jnp.int32, (_A, _A), 0)
            == lax.broadcasted_iota(jnp.int32, (_A, _A), 1))
    out_ref[...] = jnp.sum(jnp.where(diag, prod, 0.0)).reshape(1, 1)


def kernel(edge_index, joint_acts, edge_feats_u, node_feats_u,
           edge_feat_reflected_u, W_ju1, b_ju1, W_ju3, b_ju3, W_ju2, b_ju2,
           W_iu1, b_iu1, W_iu3, b_iu3, W_iu2, b_iu2):
    c_flat, cr_flat = _sc_codes(joint_acts, edge_index)
    nb = _E // _BE
    c_rows = c_flat.reshape(nb, 1, _BE)
    cr_rows = cr_flat.reshape(nb, 1, _BE)

    wfull = lambda shape: pl.BlockSpec(shape, lambda i: (0,) * len(shape))
    edge_out = pl.pallas_call(
        _edge_body,
        grid=(_E // _BE,),
        in_specs=[
            pl.BlockSpec((_BE, 3 * _DIM), lambda i: (i, 0)),
            pl.BlockSpec((_BE, 3 * _DIM), lambda i: (i, 0)),
            pl.BlockSpec((1, 1, _BE), lambda i: (i, 0, 0)),
            pl.BlockSpec((1, 1, _BE), lambda i: (i, 0, 0)),
            wfull((3 * _DIM, 32)),
            wfull((1, 32)),
            wfull((32, 32)),
            wfull((1, 32)),
            wfull((32, _A * _A)),
            wfull((_A * _A, 1)),
        ],
        out_specs=pl.BlockSpec((1, 1), lambda i: (0, 0)),
        out_shape=jax.ShapeDtypeStruct((1, 1), jnp.float32),
    )(edge_feats_u, edge_feat_reflected_u, c_rows, cr_rows,
      W_ju1, b_ju1.reshape(1, -1), W_ju3, b_ju3.reshape(1, -1),
      W_ju2, b_ju2.reshape(-1, 1))

    node_out = pl.pallas_call(
        _node_body,
        grid=(1,),
        in_specs=[
            pl.BlockSpec((_N, 2 * _DIM), lambda i: (0, 0)),
            pl.BlockSpec((1, 1, _N), lambda i: (0, 0, 0)),
            wfull((2 * _DIM, 32)),
            wfull((1, 32)),
            wfull((32, 32)),
            wfull((1, 32)),
            wfull((32, _A)),
            wfull((1, _A)),
        ],
        out_specs=pl.BlockSpec((1, 1), lambda i: (0, 0)),
        out_shape=jax.ShapeDtypeStruct((1, 1), jnp.float32),
    )(node_feats_u, joint_acts.reshape(1, 1, _N),
      W_iu1, b_iu1.reshape(1, -1), W_iu3, b_iu3.reshape(1, -1),
      W_iu2, b_iu2.reshape(1, -1))

    return node_out + 0.25 * edge_out


# speedup vs baseline: 9.7809x; 1.0913x over previous
import functools

import jax
import jax.numpy as jnp
from jax import lax
from jax.experimental import pallas as pl
from jax.experimental.pallas import tpu as pltpu
from jax.experimental.pallas import tpu_sc as plsc

_N = 10000
_E = 160000
_A = 8
_DIM = 128

_NUM_WORKERS = 32
_CHUNK = _E // _NUM_WORKERS
_CPAD = (_CHUNK + 15) // 16 * 16

_BE = 6400


def _sc_codes(joint_acts, edge_index):
    mesh = plsc.VectorSubcoreMesh(core_axis_name="c", subcore_axis_name="s")

    @functools.partial(
        pl.kernel,
        mesh=mesh,
        compiler_params=pltpu.CompilerParams(needs_layout_passes=False),
        out_type=[
            jax.ShapeDtypeStruct((_E,), jnp.int32),
            jax.ShapeDtypeStruct((_E,), jnp.int32),
        ],
        scratch_types=[
            pltpu.VMEM((_N,), jnp.int32),
            pltpu.VMEM((_CPAD,), jnp.int32),
            pltpu.VMEM((_CPAD,), jnp.int32),
            pltpu.VMEM((_CPAD,), jnp.int32),
            pltpu.VMEM((_CPAD,), jnp.int32),
        ],
    )
    def k(ja_hbm, ei_hbm, c_hbm, cr_hbm, ja_v, src_v, dst_v, c_v, cr_v):
        wid = lax.axis_index("s") * 2 + lax.axis_index("c")
        base = wid * _CHUNK
        src_v[pl.ds(_CPAD - 16, 16)] = jnp.zeros((16,), jnp.int32)
        dst_v[pl.ds(_CPAD - 16, 16)] = jnp.zeros((16,), jnp.int32)
        pltpu.sync_copy(ja_hbm, ja_v)
        pltpu.sync_copy(ei_hbm.at[pl.ds(base, _CHUNK)], src_v.at[pl.ds(0, _CHUNK)])
        pltpu.sync_copy(ei_hbm.at[pl.ds(_E + base, _CHUNK)], dst_v.at[pl.ds(0, _CHUNK)])

        def body(i, carry):
            sl = pl.ds(i * 16, 16)
            a_s = plsc.load_gather(ja_v, [src_v[sl]])
            a_d = plsc.load_gather(ja_v, [dst_v[sl]])
            c_v[sl] = a_s * _A + a_d
            cr_v[sl] = a_d * _A + a_s
            return carry

        lax.fori_loop(0, _CPAD // 16, body, 0, unroll=4)
        pltpu.sync_copy(c_v.at[pl.ds(0, _CHUNK)], c_hbm.at[pl.ds(base, _CHUNK)])
        pltpu.sync_copy(cr_v.at[pl.ds(0, _CHUNK)], cr_hbm.at[pl.ds(base, _CHUNK)])

    return k(joint_acts, edge_index.reshape(2 * _E))


def _edge_body(xu_ref, xr_ref, c_ref, cr_ref,
               w1_ref, b1_ref, w3_ref, b3_ref, w2_ref, b2t_ref, out_ref):
    @pl.when(pl.program_id(0) == 0)
    def _init():
        out_ref[...] = jnp.zeros_like(out_ref)

    diag = (lax.broadcasted_iota(jnp.int32, (_A * _A, _A * _A), 0)
            == lax.broadcasted_iota(jnp.int32, (_A * _A, _A * _A), 1))

    def stream(x, idx_row):
        g = jnp.maximum(
            jnp.dot(x, w1_ref[...], preferred_element_type=jnp.float32)
            + b1_ref[...], 0.0)
        g = jnp.maximum(
            jnp.dot(g, w3_ref[...], preferred_element_type=jnp.float32)
            + b3_ref[...], 0.0)
        onehot_t = jnp.where(
            idx_row == lax.broadcasted_iota(jnp.int32, (_A * _A, _BE), 0),
            1.0, 0.0)
        og = jnp.dot(onehot_t, g, preferred_element_type=jnp.float32)
        prod = jnp.dot(og, w2_ref[...], preferred_element_type=jnp.float32)
        return (jnp.sum(jnp.where(diag, prod, 0.0))
                + jnp.sum(onehot_t * b2t_ref[...]))

    base = pl.multiple_of(pl.program_id(0) * _BE, 128)
    part = (stream(xu_ref[...], c_ref[pl.ds(base, _BE)].reshape(1, _BE))
            + stream(xr_ref[...], cr_ref[pl.ds(base, _BE)].reshape(1, _BE)))
    out_ref[...] = out_ref[...] + part


def _node_body(x_ref, ja_ref, edge_ref, w1_ref, b1_ref, w3_ref, b3_ref,
               w2_ref, b2_ref, out_ref):
    h = jnp.maximum(
        jnp.dot(x_ref[...], w1_ref[...], preferred_element_type=jnp.float32)
        + b1_ref[...], 0.0)
    h = jnp.maximum(
        jnp.dot(h, w3_ref[...], preferred_element_type=jnp.float32)
        + b3_ref[...], 0.0)
    logit = (jnp.dot(h, w2_ref[...], preferred_element_type=jnp.float32)
             + b2_ref[...])
    onehot_t = jnp.where(
        ja_ref[...].reshape(1, _N)
        == lax.broadcasted_iota(jnp.int32, (_A, _N), 0), 1.0, 0.0)
    prod = jnp.dot(onehot_t, logit, preferred_element_type=jnp.float32)
    diag = (lax.broadcasted_iota(jnp.int32, (_A, _A), 0)
            == lax.broadcasted_iota(jnp.int32, (_A, _A), 1))
    out_ref[...] = (jnp.sum(jnp.where(diag, prod, 0.0)).reshape(1, 1)
                    + 0.25 * edge_ref[...])


def kernel(edge_index, joint_acts, edge_feats_u, node_feats_u,
           edge_feat_reflected_u, W_ju1, b_ju1, W_ju3, b_ju3, W_ju2, b_ju2,
           W_iu1, b_iu1, W_iu3, b_iu3, W_iu2, b_iu2):
    c_flat, cr_flat = _sc_codes(joint_acts, edge_index)

    wfull = lambda shape: pl.BlockSpec(shape, lambda i: (0,) * len(shape))
    edge_out = pl.pallas_call(
        _edge_body,
        grid=(_E // _BE,),
        in_specs=[
            pl.BlockSpec((_BE, 3 * _DIM), lambda i: (i, 0)),
            pl.BlockSpec((_BE, 3 * _DIM), lambda i: (i, 0)),
            pl.BlockSpec((_E,), lambda i: (0,)),
            pl.BlockSpec((_E,), lambda i: (0,)),
            wfull((3 * _DIM, 32)),
            wfull((1, 32)),
            wfull((32, 32)),
            wfull((1, 32)),
            wfull((32, _A * _A)),
            wfull((_A * _A, 1)),
        ],
        out_specs=pl.BlockSpec((1, 1), lambda i: (0, 0)),
        out_shape=jax.ShapeDtypeStruct((1, 1), jnp.float32),
    )(edge_feats_u, edge_feat_reflected_u, c_flat, cr_flat,
      W_ju1, b_ju1.reshape(1, -1), W_ju3, b_ju3.reshape(1, -1),
      W_ju2, b_ju2.reshape(-1, 1))

    node_out = pl.pallas_call(
        _node_body,
        grid=(1,),
        in_specs=[
            pl.BlockSpec((_N, 2 * _DIM), lambda i: (0, 0)),
            pl.BlockSpec((1, 1, _N), lambda i: (0, 0, 0)),
            wfull((1, 1)),
            wfull((2 * _DIM, 32)),
            wfull((1, 32)),
            wfull((32, 32)),
            wfull((1, 32)),
            wfull((32, _A)),
            wfull((1, _A)),
        ],
        out_specs=pl.BlockSpec((1, 1), lambda i: (0, 0)),
        out_shape=jax.ShapeDtypeStruct((1, 1), jnp.float32),
    )(node_feats_u, joint_acts.reshape(1, 1, _N), edge_out,
      W_iu1, b_iu1.reshape(1, -1), W_iu3, b_iu3.reshape(1, -1),
      W_iu2, b_iu2.reshape(1, -1))

    return node_out
